# Initial kernel scaffold; baseline (speedup 1.0000x reference)
import functools

import jax
import jax.numpy as jnp
from jax import lax
from jax.experimental import pallas as pl
from jax.experimental.pallas import tpu as pltpu
from jax.experimental.pallas import tpu_sc as plsc

LANES = 16
NCORES = 2
NSUB = 16
NW = NCORES * NSUB
B = 128


def _matmul_body(x_ref, w_ref, o_ref):
    o_ref[...] = jnp.dot(x_ref[...], w_ref[...],
                         preferred_element_type=jnp.float32)


def _combine_body(p_ref, b_ref, o_ref):
    o_ref[...] = p_ref[0] + p_ref[1] + b_ref[...]


def _make_spmm(n, d, g):
    rows_per_tile = n // NSUB
    chunk = rows_per_tile
    nchunks = 1
    while chunk > B:
        nchunks *= 5 if chunk % 5 == 0 else 2
        chunk = rows_per_tile // nchunks
    vecs = d // LANES

    mesh = plsc.VectorSubcoreMesh(core_axis_name="c", subcore_axis_name="s")

    @functools.partial(
        pl.kernel,
        out_type=jax.ShapeDtypeStruct((NCORES, n, d), jnp.float32),
        mesh=mesh,
        scratch_types=[
            pltpu.VMEM((g, B), jnp.int32),
            pltpu.VMEM((g, B), jnp.int32),
            pltpu.VMEM((g, B), jnp.float32),
            pltpu.VMEM((B, d), jnp.float32),
            pltpu.VMEM_SHARED((n, d), jnp.float32),
            pltpu.SemaphoreType.DMA,
        ],
    )
    def spmm(support_hbm, src_hbm, dst_hbm, w_hbm, out_hbm,
             idx_s, idx_d, wts, rows, acc, gsem):
        cid = lax.axis_index("c")
        sid = lax.axis_index("s")
        wid = cid * NSUB + sid

        pltpu.sync_copy(src_hbm.at[wid], idx_s)
        pltpu.sync_copy(dst_hbm.at[wid], idx_d)
        pltpu.sync_copy(w_hbm.at[wid], wts)

        zero = jnp.zeros((LANES,), jnp.float32)

        def zero_row(i, _):
            for j in range(vecs):
                rows[i, pl.ds(LANES * j, LANES)] = zero
            return 0

        lax.fori_loop(0, chunk, zero_row, 0)
        rbase = sid * rows_per_tile
        for k in range(nchunks):
            pltpu.sync_copy(rows.at[pl.ds(0, chunk)],
                            acc.at[pl.ds(rbase + chunk * k, chunk)])
        plsc.subcore_barrier()

        def batch_body(gi, _):
            pltpu.async_copy(support_hbm.at[idx_s.at[gi]], rows, gsem).wait()

            def scale_edge(e, _):
                wscal = wts[gi, e]
                for j in range(vecs):
                    sl = pl.ds(LANES * j, LANES)
                    rows[e, sl] = rows[e, sl] * wscal
                return 0

            lax.fori_loop(0, B, scale_edge, 0)

            pltpu.sync_copy(rows, acc.at[idx_d.at[gi]], add=True)
            return 0

        lax.fori_loop(0, g, batch_body, 0)

        plsc.subcore_barrier()
        for k in range(nchunks):
            sl = pl.ds(rbase + chunk * k, chunk)
            pltpu.sync_copy(acc.at[sl], out_hbm.at[cid, sl])

    return spmm


def kernel(x, edge_index, edge_weight, W, b):
    n, d_in = x.shape
    d = W.shape[1]
    e = edge_weight.shape[0]

    support = pl.pallas_call(
        _matmul_body,
        out_shape=jax.ShapeDtypeStruct((n, d), jnp.float32),
    )(x, W)

    per_w = -(-e // NW)
    g = -(-per_w // B)
    e_pad = NW * g * B
    pad = e_pad - e
    src = jnp.pad(edge_index[0], (0, pad)).reshape(NW, g, B)
    dst = jnp.pad(edge_index[1], (0, pad)).reshape(NW, g, B)
    wts = jnp.pad(edge_weight, (0, pad)).reshape(NW, g, B)

    partials = _make_spmm(n, d, g)(support, src, dst, wts)

    out = pl.pallas_call(
        _combine_body,
        out_shape=jax.ShapeDtypeStruct((n, d), jnp.float32),
    )(partials, b.reshape(1, d))
    return out


# speedup vs baseline: 4.8692x; 4.8692x over previous
import functools

import jax
import jax.numpy as jnp
from jax import lax
from jax.experimental import pallas as pl
from jax.experimental.pallas import tpu as pltpu
from jax.experimental.pallas import tpu_sc as plsc

LANES = 16
NCORES = 2
NSUB = 16
NW = NCORES * NSUB
B = 128


def _matmul_body(x_ref, w_ref, o_ref):
    o_ref[...] = jnp.dot(x_ref[...], w_ref[...],
                         preferred_element_type=jnp.float32)


def _combine_body(n, p_ref, b_ref, o_ref):
    o_ref[...] = p_ref[0, :n] + p_ref[1, :n] + b_ref[...]


def _make_spmm(n, d, g):
    n_pad = -(-n // (NSUB * B)) * (NSUB * B)
    rows_per_tile = n_pad // NSUB
    chunk = B
    nchunks = rows_per_tile // B
    vecs = d // LANES

    mesh = plsc.VectorSubcoreMesh(core_axis_name="c", subcore_axis_name="s")

    @functools.partial(
        pl.kernel,
        out_type=jax.ShapeDtypeStruct((NCORES, n_pad, d), jnp.float32),
        mesh=mesh,
        scratch_types=[
            pltpu.VMEM((g, B), jnp.int32),
            pltpu.VMEM((g, B), jnp.int32),
            pltpu.VMEM((g, B), jnp.float32),
            pltpu.VMEM((B, d), jnp.float32),
            pltpu.VMEM_SHARED((n_pad, d), jnp.float32),
            pltpu.SemaphoreType.DMA,
        ],
    )
    def spmm(support_hbm, src_hbm, dst_hbm, w_hbm, out_hbm,
             idx_s, idx_d, wts, rows, acc, gsem):
        cid = lax.axis_index("c")
        sid = lax.axis_index("s")
        wid = cid * NSUB + sid

        pltpu.sync_copy(src_hbm.at[wid], idx_s)
        pltpu.sync_copy(dst_hbm.at[wid], idx_d)
        pltpu.sync_copy(w_hbm.at[wid], wts)

        zero = jnp.zeros((LANES,), jnp.float32)

        def zero_row(i, _):
            for j in range(vecs):
                rows[i, pl.ds(LANES * j, LANES)] = zero
            return 0

        lax.fori_loop(0, chunk, zero_row, 0)
        rbase = sid * rows_per_tile
        for k in range(nchunks):
            pltpu.sync_copy(rows.at[pl.ds(0, chunk)],
                            acc.at[pl.ds(rbase + chunk * k, chunk)])
        plsc.subcore_barrier()

        def batch_body(gi, _):
            pltpu.async_copy(support_hbm.at[idx_s.at[gi]], rows, gsem).wait()

            def scale_group(q, _):
                base = LANES * q
                wvec = wts[gi, pl.ds(base, LANES)]
                for el in range(LANES):
                    wscal = wvec[el]
                    for j in range(vecs):
                        sl = pl.ds(LANES * j, LANES)
                        rows[base + el, sl] = rows[base + el, sl] * wscal
                return 0

            lax.fori_loop(0, B // LANES, scale_group, 0)

            pltpu.sync_copy(rows, acc.at[idx_d.at[gi]], add=True)
            return 0

        lax.fori_loop(0, g, batch_body, 0)

        plsc.subcore_barrier()
        for k in range(nchunks):
            sl = pl.ds(rbase + chunk * k, chunk)
            pltpu.sync_copy(acc.at[sl], out_hbm.at[cid, sl])

    return spmm


def kernel(x, edge_index, edge_weight, W, b):
    n, d_in = x.shape
    d = W.shape[1]
    e = edge_weight.shape[0]

    support = pl.pallas_call(
        _matmul_body,
        out_shape=jax.ShapeDtypeStruct((n, d), jnp.float32),
    )(x, W)

    per_w = -(-e // NW)
    g = -(-per_w // B)
    e_pad = NW * g * B
    pad = e_pad - e
    src = jnp.pad(edge_index[0], (0, pad)).reshape(NW, g, B)
    dst = jnp.pad(edge_index[1], (0, pad)).reshape(NW, g, B)
    wts = jnp.pad(edge_weight, (0, pad)).reshape(NW, g, B)

    partials = _make_spmm(n, d, g)(support, src, dst, wts)

    out = pl.pallas_call(
        functools.partial(_combine_body, n),
        out_shape=jax.ShapeDtypeStruct((n, d), jnp.float32),
    )(partials, b.reshape(1, d))
    return out


# speedup vs baseline: 7.0796x; 1.4539x over previous
import functools

import jax
import jax.numpy as jnp
from jax import lax
from jax.experimental import pallas as pl
from jax.experimental.pallas import tpu as pltpu
from jax.experimental.pallas import tpu_sc as plsc

LANES = 16
NCORES = 2
NSUB = 16
B = 128
NBUF = 6


def _matmul_body(n, n_pad, dh, x_ref, w_ref, o_ref):
    s = jnp.dot(x_ref[...], w_ref[...], preferred_element_type=jnp.float32)
    zeros = jnp.zeros((n_pad - n, dh), jnp.float32)
    for c in range(NCORES):
        o_ref[c, pl.ds(0, n)] = s[:, c * dh:(c + 1) * dh]
        o_ref[c, pl.ds(n, n_pad - n)] = zeros


def _combine_body(n, p_ref, b_ref, o_ref):
    o_ref[...] = jnp.concatenate([p_ref[0, :n], p_ref[1, :n]], axis=1) \
        + b_ref[...]


def _make_spmm(n_pad, dh, g):
    rows_per_tile = n_pad // NSUB
    nchunks = rows_per_tile // B
    vecs = dh // LANES

    mesh = plsc.VectorSubcoreMesh(core_axis_name="c", subcore_axis_name="s")

    @functools.partial(
        pl.kernel,
        out_type=jax.ShapeDtypeStruct((NCORES, n_pad, dh), jnp.float32),
        mesh=mesh,
        scratch_types=[
            pltpu.VMEM((NBUF, 2, B), jnp.int32),
            pltpu.VMEM((NBUF, B), jnp.float32),
            pltpu.VMEM((NBUF, B, dh), jnp.float32),
            pltpu.VMEM_SHARED((n_pad, dh), jnp.float32),
            pltpu.SemaphoreType.DMA((NBUF,)),
            pltpu.SemaphoreType.DMA((NBUF,)),
            pltpu.SemaphoreType.DMA((NBUF,)),
            pltpu.SemaphoreType.DMA((NBUF,)),
        ],
        compiler_params=pltpu.CompilerParams(use_tc_tiling_on_sc=False),
    )
    def spmm(support_hbm, edat_hbm, wdat_hbm, out_hbm, idxr, wring, rows, acc,
             isem, wsem, gsem, ssem):
        cid = lax.axis_index("c")
        sid = lax.axis_index("s")

        zero = jnp.zeros((LANES,), jnp.float32)

        def zero_row(i, _):
            for j in range(vecs):
                rows[0, i, pl.ds(LANES * j, LANES)] = zero
            return 0

        lax.fori_loop(0, B, zero_row, 0)
        rbase = sid * rows_per_tile
        for k in range(nchunks):
            pltpu.sync_copy(rows.at[0], acc.at[pl.ds(rbase + B * k, B)])
        plsc.subcore_barrier()

        def idx_start(gi, sl):
            pltpu.async_copy(edat_hbm.at[cid, sid, gi], idxr.at[sl],
                             isem.at[sl])
            pltpu.async_copy(wdat_hbm.at[sid, gi], wring.at[sl],
                             wsem.at[sl])

        def idx_wait(gi, sl):
            pltpu.make_async_copy(edat_hbm.at[cid, sid, gi], idxr.at[sl],
                                  isem.at[sl]).wait()
            pltpu.make_async_copy(wdat_hbm.at[sid, gi], wring.at[sl],
                                  wsem.at[sl]).wait()

        def gather_start(sl):
            pltpu.async_copy(support_hbm.at[idxr.at[sl, 0]], rows.at[sl],
                             gsem.at[sl])

        def gather_wait(sl):
            pltpu.make_async_copy(support_hbm.at[idxr.at[sl, 0]],
                                  rows.at[sl], gsem.at[sl]).wait()

        def scatter_start(sl):
            pltpu.async_copy(rows.at[sl], acc.at[idxr.at[sl, 1]],
                             ssem.at[sl], add=True)

        def scatter_wait(sl):
            pltpu.make_async_copy(rows.at[sl], acc.at[idxr.at[sl, 1]],
                                  ssem.at[sl]).wait()

        def scale(sl):
            def scale_group(q, _):
                base = LANES * q
                wvec = wring[sl, pl.ds(base, LANES)]
                for el in range(LANES):
                    wscal = wvec[el]
                    for j in range(vecs):
                        c = pl.ds(LANES * j, LANES)
                        rows[sl, base + el, c] = rows[sl, base + el, c] * wscal
                return 0

            lax.fori_loop(0, B // LANES, scale_group, 0)

        def step(gi, k):
            @pl.when(gi + 2 < g)
            def _():
                idx_wait(gi + 2, (k + 2) % NBUF)
                gather_start((k + 2) % NBUF)

            gather_wait(k)
            scale(k)
            s_prev = (k + NBUF - 1) % NBUF

            @pl.when(gi >= 1)
            def _():
                scatter_wait(s_prev)

            @pl.when(gi + NBUF - 1 < g)
            def _():
                idx_start(gi + NBUF - 1, s_prev)

            scatter_start(k)

        for b0 in range(min(NBUF - 1, g)):
            idx_start(b0, b0)
        for b0 in range(min(2, g)):
            idx_wait(b0, b0)
            gather_start(b0)

        g_main = g // NBUF * NBUF

        def outer(t, _):
            for k in range(NBUF):
                step(t * NBUF + k, k)
            return 0

        lax.fori_loop(0, g_main // NBUF, outer, 0)
        for gi in range(g_main, g):
            step(jnp.int32(gi), gi % NBUF)
        scatter_wait((g - 1) % NBUF)

        plsc.subcore_barrier()
        for k in range(nchunks):
            sl = pl.ds(rbase + B * k, B)
            pltpu.sync_copy(acc.at[sl], out_hbm.at[cid, sl])

    return spmm


def kernel(x, edge_index, edge_weight, W, b):
    n, d_in = x.shape
    d = W.shape[1]
    dh = d // NCORES
    e = edge_weight.shape[0]
    n_pad = -(-n // (NSUB * B)) * (NSUB * B)

    support = pl.pallas_call(
        functools.partial(_matmul_body, n, n_pad, dh),
        out_shape=jax.ShapeDtypeStruct((NCORES, n_pad, dh), jnp.float32),
    )(x, W)

    per_t = -(-e // NSUB)
    g = -(-per_t // B)
    pad = NSUB * g * B - e
    src = jnp.pad(edge_index[0], (0, pad)).reshape(NSUB, g, 1, B)
    dst = jnp.pad(edge_index[1], (0, pad)).reshape(NSUB, g, 1, B)
    wdat = jnp.pad(edge_weight, (0, pad)).reshape(NSUB, g, B)
    base = jnp.concatenate([src, dst], axis=2)
    off = jnp.zeros((NCORES, 1, 1, 2, 1), jnp.int32).at[1, 0, 0, 0, 0].set(
        n_pad)
    edat = base[None] + off

    partials = _make_spmm(n_pad, dh, g)(
        support.reshape(NCORES * n_pad, dh), edat, wdat)

    out = pl.pallas_call(
        functools.partial(_combine_body, n),
        out_shape=jax.ShapeDtypeStruct((n, d), jnp.float32),
    )(partials, b.reshape(1, d))
    return out


# speedup vs baseline: 7.2956x; 1.0305x over previous
import functools

import jax
import jax.numpy as jnp
from jax import lax
from jax.experimental import pallas as pl
from jax.experimental.pallas import tpu as pltpu
from jax.experimental.pallas import tpu_sc as plsc

LANES = 16
NCORES = 2
NSUB = 16
B = 128
NBUF = 6


def _matmul_body(n, n_pad, dh, x_ref, w_ref, o_ref):
    s = jnp.dot(x_ref[...], w_ref[...], preferred_element_type=jnp.float32)
    zeros = jnp.zeros((n_pad - n, dh), jnp.float32)
    for c in range(NCORES):
        o_ref[c, pl.ds(0, n)] = s[:, c * dh:(c + 1) * dh]
        o_ref[c, pl.ds(n, n_pad - n)] = zeros


def _combine_body(n, p_ref, b_ref, o_ref):
    o_ref[...] = jnp.concatenate([p_ref[0, :n], p_ref[1, :n]], axis=1) \
        + b_ref[...]


def _make_spmm(n_pad, dh, g):
    rows_per_tile = n_pad // NSUB
    nchunks = rows_per_tile // B
    vecs = dh // LANES

    mesh = plsc.VectorSubcoreMesh(core_axis_name="c", subcore_axis_name="s")

    @functools.partial(
        pl.kernel,
        out_type=jax.ShapeDtypeStruct((NCORES, n_pad, dh), jnp.float32),
        mesh=mesh,
        scratch_types=[
            pltpu.VMEM((NBUF, B), jnp.int32),
            pltpu.VMEM((NBUF, B), jnp.int32),
            pltpu.VMEM((NBUF, B), jnp.float32),
            pltpu.VMEM((NBUF, B, dh), jnp.float32),
            pltpu.VMEM_SHARED((n_pad, dh), jnp.float32),
            pltpu.SemaphoreType.DMA((NBUF,)),
            pltpu.SemaphoreType.DMA((NBUF,)),
            pltpu.SemaphoreType.DMA((NBUF,)),
            pltpu.SemaphoreType.DMA((NBUF,)),
            pltpu.SemaphoreType.DMA((NBUF,)),
        ],
        compiler_params=pltpu.CompilerParams(use_tc_tiling_on_sc=False),
    )
    def spmm(support_hbm, src_hbm, dst_hbm, w_hbm, out_hbm,
             sring, dring, wring, rows, acc, isem, dsem, wsem, gsem, ssem):
        cid = lax.axis_index("c")
        sid = lax.axis_index("s")
        srcoff = cid * n_pad

        zero = jnp.zeros((LANES,), jnp.float32)

        def zero_row(i, _):
            for j in range(vecs):
                rows[0, i, pl.ds(LANES * j, LANES)] = zero
            return 0

        lax.fori_loop(0, B, zero_row, 0)
        rbase = sid * rows_per_tile
        for k in range(nchunks):
            pltpu.sync_copy(rows.at[0], acc.at[pl.ds(rbase + B * k, B)])
        plsc.subcore_barrier()

        def idx_start(gi, sl):
            pltpu.async_copy(src_hbm.at[sid, gi], sring.at[sl], isem.at[sl])
            pltpu.async_copy(dst_hbm.at[sid, gi], dring.at[sl], dsem.at[sl])
            pltpu.async_copy(w_hbm.at[sid, gi], wring.at[sl], wsem.at[sl])

        def idx_wait(gi, sl):
            pltpu.make_async_copy(src_hbm.at[sid, gi], sring.at[sl],
                                  isem.at[sl]).wait()
            pltpu.make_async_copy(dst_hbm.at[sid, gi], dring.at[sl],
                                  dsem.at[sl]).wait()
            pltpu.make_async_copy(w_hbm.at[sid, gi], wring.at[sl],
                                  wsem.at[sl]).wait()

        def add_srcoff(sl):
            for q in range(B // LANES):
                s = pl.ds(LANES * q, LANES)
                sring[sl, s] = sring[sl, s] + srcoff

        def gather_start(sl):
            pltpu.async_copy(support_hbm.at[sring.at[sl]], rows.at[sl],
                             gsem.at[sl])

        def gather_wait(sl):
            pltpu.make_async_copy(support_hbm.at[sring.at[sl]],
                                  rows.at[sl], gsem.at[sl]).wait()

        def scatter_start(sl):
            pltpu.async_copy(rows.at[sl], acc.at[dring.at[sl]],
                             ssem.at[sl], add=True)

        def scatter_wait(sl):
            pltpu.make_async_copy(rows.at[sl], acc.at[dring.at[sl]],
                                  ssem.at[sl]).wait()

        def scale(sl):
            def scale_group(q, _):
                base = LANES * q
                wvec = wring[sl, pl.ds(base, LANES)]
                for el in range(LANES):
                    wscal = wvec[el]
                    for j in range(vecs):
                        c = pl.ds(LANES * j, LANES)
                        rows[sl, base + el, c] = rows[sl, base + el, c] * wscal
                return 0

            lax.fori_loop(0, B // LANES, scale_group, 0)

        def step(gi, k):
            @pl.when(gi + 2 < g)
            def _():
                idx_wait(gi + 2, (k + 2) % NBUF)
                add_srcoff((k + 2) % NBUF)
                gather_start((k + 2) % NBUF)

            gather_wait(k)
            scale(k)
            s_prev = (k + NBUF - 1) % NBUF

            @pl.when(gi >= 1)
            def _():
                scatter_wait(s_prev)

            @pl.when(gi + NBUF - 1 < g)
            def _():
                idx_start(gi + NBUF - 1, s_prev)

            scatter_start(k)

        for b0 in range(min(NBUF - 1, g)):
            idx_start(b0, b0)
        for b0 in range(min(2, g)):
            idx_wait(b0, b0)
            add_srcoff(b0)
            gather_start(b0)

        g_main = g // NBUF * NBUF

        def outer(t, _):
            for k in range(NBUF):
                step(t * NBUF + k, k)
            return 0

        lax.fori_loop(0, g_main // NBUF, outer, 0)
        for gi in range(g_main, g):
            step(jnp.int32(gi), gi % NBUF)
        scatter_wait((g - 1) % NBUF)

        plsc.subcore_barrier()
        for k in range(nchunks):
            sl = pl.ds(rbase + B * k, B)
            pltpu.sync_copy(acc.at[sl], out_hbm.at[cid, sl])

    return spmm


def kernel(x, edge_index, edge_weight, W, b):
    n, d_in = x.shape
    d = W.shape[1]
    dh = d // NCORES
    e = edge_weight.shape[0]
    n_pad = -(-n // (NSUB * B)) * (NSUB * B)

    support = pl.pallas_call(
        functools.partial(_matmul_body, n, n_pad, dh),
        out_shape=jax.ShapeDtypeStruct((NCORES, n_pad, dh), jnp.float32),
    )(x, W)

    per_t = -(-e // NSUB)
    g = -(-per_t // B)
    pad = NSUB * g * B - e
    src = jnp.pad(edge_index[0], (0, pad)).reshape(NSUB, g, B)
    dst = jnp.pad(edge_index[1], (0, pad)).reshape(NSUB, g, B)
    wdat = jnp.pad(edge_weight, (0, pad)).reshape(NSUB, g, B)

    partials = _make_spmm(n_pad, dh, g)(
        support.reshape(NCORES * n_pad, dh), src, dst, wdat)

    out = pl.pallas_call(
        functools.partial(_combine_body, n),
        out_shape=jax.ShapeDtypeStruct((n, d), jnp.float32),
    )(partials, b.reshape(1, d))
    return out
